# Initial kernel scaffold; baseline (speedup 1.0000x reference)
import jax
import jax.numpy as jnp
from jax.experimental import pallas as pl


def kernel(x, proposals, W1, b1, W2, b2, Wc, bc, Wb, bb):
    raise NotImplementedError("write your pallas kernel here")


# speedup vs baseline: 33.5107x; 33.5107x over previous
import math

import jax
import jax.numpy as jnp
from jax.experimental import pallas as pl
from jax.experimental.pallas import tpu as pltpu

_SCORE_THRESH = 0.05
_NMS_THRESH = 0.5
_DET = 100
_NCLS = 81
_NFG = 80
_PRE = 2000
_IMG_H, _IMG_W = 800.0, 1333.0
_CLIP = math.log(1000.0 / 16.0)

_N = 20000
_NPAD = 20480
_TN = 256
_NTILES = _NPAD // _TN
_CH = 2048
_NCHUNK = (_NPAD * _NFG) // _CH
_R = 8

_INTERPRET = False


def _rep(x):
    return jnp.broadcast_to(x, (_R,) + x.shape[1:])


def _cmp_stage(lane, k, j, s, i, payloads, descending):
    pplus = (lane & j) == 0
    def part(a):
        return jnp.where(pplus, jnp.roll(a, -j, axis=1), jnp.roll(a, j, axis=1))
    ps, pi = part(s), part(i)
    better = (s > ps) | ((s == ps) & (i < pi))
    if not descending:
        better = ~better
    dirblock = (lane & k) == 0
    keep_self = dirblock ^ pplus ^ better
    s2 = jnp.where(keep_self, s, ps)
    i2 = jnp.where(keep_self, i, pi)
    pay2 = [jnp.where(keep_self, p, part(p)) for p in payloads]
    return s2, i2, pay2


def _bitonic_sort(s, i, payloads, descending=True):
    n = s.shape[1]
    lane = jax.lax.broadcasted_iota(jnp.int32, (_R, n), 1)
    k = 2
    while k <= n:
        j = k // 2
        while j >= 1:
            s, i, payloads = _cmp_stage(lane, k, j, s, i, payloads, descending)
            j //= 2
        k *= 2
    return s, i, payloads


def _bitonic_merge_desc(s, i, payloads):
    n = s.shape[1]
    lane = jax.lax.broadcasted_iota(jnp.int32, (_R, n), 1)
    j = n // 2
    while j >= 1:
        s, i, payloads = _cmp_stage(lane, n, j, s, i, payloads, True)
        j //= 2
    return s, i, payloads


def _num_unsorted_desc(s):
    return jnp.sum(jnp.where(s[:, :-1] < s[:, 1:], 1.0, 0.0))


def _head_body(x_ref, p_ref, w1_ref, b1_ref, w2_ref, b2_ref, wc_ref, bc_ref,
               wb_ref, bb_ref, perm_ref, s_ref, bx_ref):
    f1 = jnp.maximum(
        jnp.dot(x_ref[...], w1_ref[...], preferred_element_type=jnp.float32)
        + b1_ref[...], 0.0)
    f2 = jnp.maximum(
        jnp.dot(f1, w2_ref[...], preferred_element_type=jnp.float32)
        + b2_ref[...], 0.0)
    logits = (jnp.dot(f2, wc_ref[...], preferred_element_type=jnp.float32)
              + bc_ref[...])
    col = jax.lax.broadcasted_iota(jnp.int32, (_TN, 128), 1)
    logits = jnp.where(col < _NCLS, logits, -1e30)
    m = jnp.max(logits, axis=-1, keepdims=True)
    e = jnp.exp(logits - m)
    sm = e / jnp.sum(e, axis=-1, keepdims=True)
    sfg = sm[:, 1:_NCLS]

    breg = (jnp.dot(f2, wb_ref[...], preferred_element_type=jnp.float32)
            + bb_ref[...])
    dx = breg[:, 1:81] / 10.0
    dy = breg[:, 129:209] / 10.0
    dw = jnp.minimum(breg[:, 257:337] / 5.0, _CLIP)
    dh = jnp.minimum(breg[:, 385:465] / 5.0, _CLIP)

    px1 = p_ref[:, 0:1]
    py1 = p_ref[:, 1:2]
    px2 = p_ref[:, 2:3]
    py2 = p_ref[:, 3:4]
    w = px2 - px1 + 1.0
    h = py2 - py1 + 1.0
    cx = px1 + 0.5 * w
    cy = py1 + 0.5 * h
    pcx = dx * w + cx
    pcy = dy * h + cy
    pw = jnp.exp(dw) * w
    ph = jnp.exp(dh) * h
    bx1 = jnp.clip(pcx - 0.5 * pw, 0.0, _IMG_W - 1.0)
    by1 = jnp.clip(pcy - 0.5 * ph, 0.0, _IMG_H - 1.0)
    bx2 = jnp.clip(pcx + 0.5 * pw - 1.0, 0.0, _IMG_W - 1.0)
    by2 = jnp.clip(pcy + 0.5 * ph - 1.0, 0.0, _IMG_H - 1.0)

    row = (jax.lax.broadcasted_iota(jnp.int32, (_TN, _NFG), 0)
           + pl.program_id(0) * _TN)
    msk = jnp.where(sfg > _SCORE_THRESH, sfg, -1.0)
    msk = jnp.where(row < _N, msk, -2.0)
    s_ref[...] = msk

    planes = jnp.concatenate([bx1, by1, bx2, by2], axis=1)
    bx_ref[...] = jnp.dot(planes, perm_ref[...],
                          preferred_element_type=jnp.float32)


def _post_body(s_ref, boxes_ref, obox_ref, osc_ref, olb_ref,
               bs, bi, keep, rows4, obox8, ffs, ffl, ffl_sm, bi_sm, sem):
    lane = jax.lax.broadcasted_iota(jnp.int32, (_R, _CH), 1)
    c0 = _rep(s_ref[0:1, :])

    unsorted0 = _num_unsorted_desc(c0)

    @pl.when(unsorted0 == 0.0)
    def _():
        bs[...] = c0
        bi[...] = lane

    @pl.when(unsorted0 != 0.0)
    def _():
        ss, ii, _ = _bitonic_sort(c0, lane, [])
        bs[...] = ss
        bi[...] = ii

    bmin0 = bs[0, _CH - 1]
    full_max = jnp.max(s_ref[...])

    @pl.when(full_max > bmin0)
    def _():
        def merge_body(c, _):
            ch = _rep(s_ref[pl.ds(c, 1), :])
            m = jnp.max(ch)

            @pl.when(m > bmin0)
            def _():
                ci = lane + c * _CH
                sa, ia, _ = _bitonic_sort(ch, ci, [], descending=False)
                cs = jnp.concatenate([bs[...], sa], axis=1)
                cidx = jnp.concatenate([bi[...], ia], axis=1)
                ms, mi, _ = _bitonic_merge_desc(cs, cidx, [])
                bs[...] = ms[:, :_CH]
                bi[...] = mi[:, :_CH]
            return 0

        jax.lax.fori_loop(1, _NCHUNK, merge_body, 0)

    pc_f = jnp.sum(jnp.where(
        (lane < _PRE) & (bs[...] > _SCORE_THRESH), 1.0, 0.0)) / _R
    pc = pc_f.astype(jnp.int32)

    keep[...] = jnp.ones((_R, _CH), jnp.int32)

    @pl.when(pc > 0)
    def _():
        rows4[...] = jnp.zeros((_CH, 128), jnp.float32)
        cp = pltpu.make_async_copy(bi.at[0:1, :], bi_sm, sem)
        cp.start()
        cp.wait()

        def issue(j, _):
            flat = bi_sm[0, j]
            n = flat // _NFG
            cc = flat % _NFG
            base = pl.multiple_of(128 * ((4 * cc) // 128), 128)
            pltpu.make_async_copy(
                boxes_ref.at[pl.ds(n, 1), pl.ds(base, 128)],
                rows4.at[pl.ds(j, 1), :], sem).start()
            return 0

        jax.lax.fori_loop(0, pc, issue, 0)

        def drain(j, _):
            pltpu.make_async_copy(
                boxes_ref.at[pl.ds(0, 1), pl.ds(0, 128)],
                rows4.at[pl.ds(0, 1), :], sem).wait()
            return 0

        jax.lax.fori_loop(0, pc, drain, 0)

        qo = ((4 * (bi[...] % _NFG)) % 128).astype(jnp.float32)
        qoT = jnp.transpose(qo[0:1, :])
        colw = jax.lax.broadcasted_iota(
            jnp.int32, (_CH, 128), 1).astype(jnp.float32)

        def coord(p):
            v = jnp.sum(
                jnp.where(colw == qoT + p, rows4[...], 0.0),
                axis=1, keepdims=True)
            return _rep(jnp.transpose(v))

        off = ((bi[...] % _NFG) + 1).astype(jnp.float32) * (_IMG_W + _IMG_H)
        cx1 = coord(0) + off
        cy1 = coord(1) + off
        cx2 = coord(2) + off
        cy2 = coord(3) + off
        areas = (cx2 - cx1 + 1.0) * (cy2 - cy1 + 1.0)

        def nms_body(i, carry):
            cx1v, cy1v, cx2v, cy2v, areasv = carry
            sel = (lane == i)
            ki = jnp.max(jnp.where(sel, keep[...], 0))
            x1i = jnp.max(jnp.where(sel, cx1v, -1e30))
            y1i = jnp.max(jnp.where(sel, cy1v, -1e30))
            x2i = jnp.max(jnp.where(sel, cx2v, -1e30))
            y2i = jnp.max(jnp.where(sel, cy2v, -1e30))
            ai = jnp.max(jnp.where(sel, areasv, -1e30))
            xx1 = jnp.maximum(x1i, cx1v)
            yy1 = jnp.maximum(y1i, cy1v)
            xx2 = jnp.minimum(x2i, cx2v)
            yy2 = jnp.minimum(y2i, cy2v)
            ww = jnp.maximum(xx2 - xx1 + 1.0, 0.0)
            hh = jnp.maximum(yy2 - yy1 + 1.0, 0.0)
            inter = ww * hh
            iou = inter / (ai + areasv - inter)
            sup = (iou > _NMS_THRESH) & (lane > i)
            keep[...] = jnp.where((ki > 0) & sup, 0, keep[...])
            return carry

        jax.lax.fori_loop(0, pc, nms_body, (cx1, cy1, cx2, cy2, areas))

    fv = jnp.where(
        lane < _PRE,
        jnp.where((bs[...] > _SCORE_THRESH) & (keep[...] > 0), bs[...], -1.0),
        -2.0)

    unsortedf = _num_unsorted_desc(fv)

    @pl.when(unsortedf == 0.0)
    def _():
        ffs[...] = fv
        ffl[...] = bi[...]

    @pl.when(unsortedf != 0.0)
    def _():
        fs2, _, (fl2,) = _bitonic_sort(fv, lane, [bi[...]])
        ffs[...] = fs2
        ffl[...] = fl2

    osc_ref[...] = ffs[0:1, 0:128]
    lab = (ffl[...] % _NFG) + 1
    olb_ref[...] = lab[0:1, 0:128]

    cp2 = pltpu.make_async_copy(ffl.at[0:1, :], ffl_sm, sem)
    cp2.start()
    cp2.wait()

    def issue_out(j, _):
        flat = ffl_sm[0, j]
        n = flat // _NFG
        cc = flat % _NFG
        base = pl.multiple_of(128 * ((4 * cc) // 128), 128)
        pltpu.make_async_copy(
            boxes_ref.at[pl.ds(n, 1), pl.ds(base, 128)],
            obox8.at[pl.ds(j, 1), :], sem).start()
        return 0

    jax.lax.fori_loop(0, _DET, issue_out, 0)

    def drain_out(j, _):
        pltpu.make_async_copy(
            boxes_ref.at[pl.ds(0, 1), pl.ds(0, 128)],
            obox8.at[pl.ds(0, 1), :], sem).wait()
        return 0

    jax.lax.fori_loop(0, _DET, drain_out, 0)

    qof = ((4 * (ffl[...] % _NFG)) % 128).astype(jnp.float32)
    qofT = jnp.transpose(qof[0:1, 0:128])
    colo = jax.lax.broadcasted_iota(
        jnp.int32, (128, 128), 1).astype(jnp.float32)
    vals = [jnp.sum(jnp.where(colo == qofT + p, obox8[...], 0.0),
                    axis=1, keepdims=True) for p in range(4)]
    obox_ref[...] = jnp.concatenate(vals, axis=1)


@jax.jit
def kernel(x, proposals, W1, b1, W2, b2, Wc, bc, Wb, bb):
    xp = jnp.pad(x, ((0, _NPAD - _N), (0, 0)))
    pp = jnp.pad(proposals, ((0, _NPAD - _N), (0, 0)))
    wcp = jnp.pad(Wc, ((0, 0), (0, 128 - _NCLS)))
    bcp = jnp.pad(bc, ((0, 128 - _NCLS),)).reshape(1, 128)
    wbp = jnp.pad(Wb.reshape(1024, _NCLS, 4).transpose(0, 2, 1),
                  ((0, 0), (0, 0), (0, 128 - _NCLS))).reshape(1024, 512)
    bbp = jnp.pad(bb.reshape(_NCLS, 4).T,
                  ((0, 0), (0, 128 - _NCLS))).reshape(1, 512)
    src = jax.lax.broadcasted_iota(jnp.int32, (320, 384), 0)
    dst = jax.lax.broadcasted_iota(jnp.int32, (320, 384), 1)
    perm = ((dst < 320) & ((dst % 4) * _NFG + dst // 4 == src)
            ).astype(jnp.float32)

    scores, boxes = pl.pallas_call(
        _head_body,
        grid=(_NTILES,),
        in_specs=[
            pl.BlockSpec((_TN, 256), lambda i: (i, 0)),
            pl.BlockSpec((_TN, 4), lambda i: (i, 0)),
            pl.BlockSpec((256, 1024), lambda i: (0, 0)),
            pl.BlockSpec((1, 1024), lambda i: (0, 0)),
            pl.BlockSpec((1024, 1024), lambda i: (0, 0)),
            pl.BlockSpec((1, 1024), lambda i: (0, 0)),
            pl.BlockSpec((1024, 128), lambda i: (0, 0)),
            pl.BlockSpec((1, 128), lambda i: (0, 0)),
            pl.BlockSpec((1024, 512), lambda i: (0, 0)),
            pl.BlockSpec((1, 512), lambda i: (0, 0)),
            pl.BlockSpec((320, 384), lambda i: (0, 0)),
        ],
        out_specs=[
            pl.BlockSpec((_TN, _NFG), lambda i: (i, 0)),
            pl.BlockSpec((_TN, 384), lambda i: (i, 0)),
        ],
        out_shape=[
            jax.ShapeDtypeStruct((_NPAD, _NFG), jnp.float32),
            jax.ShapeDtypeStruct((_NPAD, 384), jnp.float32),
        ],
        compiler_params=pltpu.CompilerParams(
            dimension_semantics=("arbitrary",)),
        interpret=_INTERPRET,
    )(xp, pp, W1, b1.reshape(1, 1024), W2, b2.reshape(1, 1024),
      wcp, bcp, wbp, bbp, perm)

    s2 = scores.reshape(_NCHUNK, _CH)

    obox, osc, olb = pl.pallas_call(
        _post_body,
        in_specs=[
            pl.BlockSpec(memory_space=pltpu.VMEM),
            pl.BlockSpec(memory_space=pltpu.HBM),
        ],
        out_specs=[
            pl.BlockSpec(memory_space=pltpu.VMEM),
            pl.BlockSpec(memory_space=pltpu.VMEM),
            pl.BlockSpec(memory_space=pltpu.VMEM),
        ],
        out_shape=[
            jax.ShapeDtypeStruct((128, 4), jnp.float32),
            jax.ShapeDtypeStruct((1, 128), jnp.float32),
            jax.ShapeDtypeStruct((1, 128), jnp.int32),
        ],
        scratch_shapes=[
            pltpu.VMEM((_R, _CH), jnp.float32),
            pltpu.VMEM((_R, _CH), jnp.int32),
            pltpu.VMEM((_R, _CH), jnp.int32),
            pltpu.VMEM((_CH, 128), jnp.float32),
            pltpu.VMEM((128, 128), jnp.float32),
            pltpu.VMEM((_R, _CH), jnp.float32),
            pltpu.VMEM((_R, _CH), jnp.int32),
            pltpu.SMEM((1, _CH), jnp.int32),
            pltpu.SMEM((1, _CH), jnp.int32),
            pltpu.SemaphoreType.DMA,
        ],
        interpret=_INTERPRET,
    )(s2, boxes)

    det_boxes = obox[:_DET, :]
    det_scores = osc[0, :_DET]
    det_labels = olb[0, :_DET]
    return det_boxes, det_scores, det_labels


# speedup vs baseline: 36.2417x; 1.0815x over previous
import math

import jax
import jax.numpy as jnp
from jax.experimental import pallas as pl
from jax.experimental.pallas import tpu as pltpu

_SCORE_THRESH = 0.05
_NMS_THRESH = 0.5
_DET = 100
_NCLS = 81
_NFG = 80
_PRE = 2000
_IMG_H, _IMG_W = 800.0, 1333.0
_CLIP = math.log(1000.0 / 16.0)

_N = 20000
_NPAD = 20480
_TN = 256
_NTILES = _NPAD // _TN
_CH = 2048
_NCHUNK = (_NPAD * _NFG) // _CH
_R = 8

_INTERPRET = False


def _rep(x):
    return jnp.broadcast_to(x, (_R,) + x.shape[1:])


def _cmp_stage(lane, k, j, s, i, payloads, descending):
    pplus = (lane & j) == 0
    def part(a):
        return jnp.where(pplus, jnp.roll(a, -j, axis=1), jnp.roll(a, j, axis=1))
    ps, pi = part(s), part(i)
    better = (s > ps) | ((s == ps) & (i < pi))
    if not descending:
        better = ~better
    dirblock = (lane & k) == 0
    keep_self = dirblock ^ pplus ^ better
    s2 = jnp.where(keep_self, s, ps)
    i2 = jnp.where(keep_self, i, pi)
    pay2 = [jnp.where(keep_self, p, part(p)) for p in payloads]
    return s2, i2, pay2


def _bitonic_sort(s, i, payloads, descending=True):
    n = s.shape[1]
    lane = jax.lax.broadcasted_iota(jnp.int32, (_R, n), 1)
    k = 2
    while k <= n:
        j = k // 2
        while j >= 1:
            s, i, payloads = _cmp_stage(lane, k, j, s, i, payloads, descending)
            j //= 2
        k *= 2
    return s, i, payloads


def _bitonic_merge_desc(s, i, payloads):
    n = s.shape[1]
    lane = jax.lax.broadcasted_iota(jnp.int32, (_R, n), 1)
    j = n // 2
    while j >= 1:
        s, i, payloads = _cmp_stage(lane, n, j, s, i, payloads, True)
        j //= 2
    return s, i, payloads


def _num_unsorted_desc(s):
    return jnp.sum(jnp.where(s[:, :-1] < s[:, 1:], 1.0, 0.0))


def _head_body(x_ref, p_ref, w1_ref, b1_ref, w2_ref, b2_ref, wc_ref, bc_ref,
               wb_ref, bb_ref, perm_ref, s_ref, bx_ref):
    f1 = jnp.maximum(
        jnp.dot(x_ref[...], w1_ref[...], preferred_element_type=jnp.float32)
        + b1_ref[...], 0.0)
    f1b = f1.astype(jnp.bfloat16)
    f2 = jnp.maximum(
        jnp.dot(f1b, w2_ref[...], preferred_element_type=jnp.float32)
        + b2_ref[...], 0.0)
    f2b = f2.astype(jnp.bfloat16)
    logits = (jnp.dot(f2b, wc_ref[...], preferred_element_type=jnp.float32)
              + bc_ref[...])
    col = jax.lax.broadcasted_iota(jnp.int32, (_TN, 128), 1)
    logits = jnp.where(col < _NCLS, logits, -1e30)
    m = jnp.max(logits, axis=-1, keepdims=True)
    e = jnp.exp(logits - m)
    sm = e / jnp.sum(e, axis=-1, keepdims=True)
    sfg = sm[:, 1:_NCLS]

    breg = (jnp.dot(f2b, wb_ref[...], preferred_element_type=jnp.float32)
            + bb_ref[...])
    dx = breg[:, 1:81] / 10.0
    dy = breg[:, 129:209] / 10.0
    dw = jnp.minimum(breg[:, 257:337] / 5.0, _CLIP)
    dh = jnp.minimum(breg[:, 385:465] / 5.0, _CLIP)

    px1 = p_ref[:, 0:1]
    py1 = p_ref[:, 1:2]
    px2 = p_ref[:, 2:3]
    py2 = p_ref[:, 3:4]
    w = px2 - px1 + 1.0
    h = py2 - py1 + 1.0
    cx = px1 + 0.5 * w
    cy = py1 + 0.5 * h
    pcx = dx * w + cx
    pcy = dy * h + cy
    pw = jnp.exp(dw) * w
    ph = jnp.exp(dh) * h
    bx1 = jnp.clip(pcx - 0.5 * pw, 0.0, _IMG_W - 1.0)
    by1 = jnp.clip(pcy - 0.5 * ph, 0.0, _IMG_H - 1.0)
    bx2 = jnp.clip(pcx + 0.5 * pw - 1.0, 0.0, _IMG_W - 1.0)
    by2 = jnp.clip(pcy + 0.5 * ph - 1.0, 0.0, _IMG_H - 1.0)

    row = (jax.lax.broadcasted_iota(jnp.int32, (_TN, _NFG), 0)
           + pl.program_id(0) * _TN)
    msk = jnp.where(sfg > _SCORE_THRESH, sfg, -1.0)
    msk = jnp.where(row < _N, msk, -2.0)
    s_ref[...] = msk

    planes = jnp.concatenate([bx1, by1, bx2, by2], axis=1)
    bx_ref[...] = jnp.dot(planes, perm_ref[...],
                          preferred_element_type=jnp.float32)


def _post_body(s_ref, boxes_ref, obox_ref, osc_ref, olb_ref,
               bs, bi, keep, rows4, obox8, ffs, ffl, ffl_sm, bi_sm, sem):
    lane = jax.lax.broadcasted_iota(jnp.int32, (_R, _CH), 1)
    c0 = _rep(s_ref[0:1, :])

    unsorted0 = _num_unsorted_desc(c0)

    @pl.when(unsorted0 == 0.0)
    def _():
        bs[...] = c0
        bi[...] = lane

    @pl.when(unsorted0 != 0.0)
    def _():
        ss, ii, _ = _bitonic_sort(c0, lane, [])
        bs[...] = ss
        bi[...] = ii

    bmin0 = bs[0, _CH - 1]
    full_max = jnp.max(s_ref[...])

    @pl.when(full_max > bmin0)
    def _():
        def merge_body(c, _):
            ch = _rep(s_ref[pl.ds(c, 1), :])
            m = jnp.max(ch)

            @pl.when(m > bmin0)
            def _():
                ci = lane + c * _CH
                sa, ia, _ = _bitonic_sort(ch, ci, [], descending=False)
                cs = jnp.concatenate([bs[...], sa], axis=1)
                cidx = jnp.concatenate([bi[...], ia], axis=1)
                ms, mi, _ = _bitonic_merge_desc(cs, cidx, [])
                bs[...] = ms[:, :_CH]
                bi[...] = mi[:, :_CH]
            return 0

        jax.lax.fori_loop(1, _NCHUNK, merge_body, 0)

    pc_f = jnp.sum(jnp.where(
        (lane < _PRE) & (bs[...] > _SCORE_THRESH), 1.0, 0.0)) / _R
    pc = pc_f.astype(jnp.int32)

    keep[...] = jnp.ones((_R, _CH), jnp.int32)

    @pl.when(pc > 0)
    def _():
        rows4[...] = jnp.zeros((_CH, 128), jnp.float32)
        cp = pltpu.make_async_copy(bi.at[0:1, :], bi_sm, sem)
        cp.start()
        cp.wait()

        def issue(j, _):
            flat = bi_sm[0, j]
            n = flat // _NFG
            cc = flat % _NFG
            base = pl.multiple_of(128 * ((4 * cc) // 128), 128)
            pltpu.make_async_copy(
                boxes_ref.at[pl.ds(n, 1), pl.ds(base, 128)],
                rows4.at[pl.ds(j, 1), :], sem).start()
            return 0

        jax.lax.fori_loop(0, pc, issue, 0)

        def drain(j, _):
            pltpu.make_async_copy(
                boxes_ref.at[pl.ds(0, 1), pl.ds(0, 128)],
                rows4.at[pl.ds(0, 1), :], sem).wait()
            return 0

        jax.lax.fori_loop(0, pc, drain, 0)

        qo = ((4 * (bi[...] % _NFG)) % 128).astype(jnp.float32)
        qoT = jnp.transpose(qo[0:1, :])
        colw = jax.lax.broadcasted_iota(
            jnp.int32, (_CH, 128), 1).astype(jnp.float32)

        def coord(p):
            v = jnp.sum(
                jnp.where(colw == qoT + p, rows4[...], 0.0),
                axis=1, keepdims=True)
            return _rep(jnp.transpose(v))

        off = ((bi[...] % _NFG) + 1).astype(jnp.float32) * (_IMG_W + _IMG_H)
        cx1 = coord(0) + off
        cy1 = coord(1) + off
        cx2 = coord(2) + off
        cy2 = coord(3) + off
        areas = (cx2 - cx1 + 1.0) * (cy2 - cy1 + 1.0)

        def nms_body(i, carry):
            cx1v, cy1v, cx2v, cy2v, areasv = carry
            sel = (lane == i)
            ki = jnp.max(jnp.where(sel, keep[...], 0))
            x1i = jnp.max(jnp.where(sel, cx1v, -1e30))
            y1i = jnp.max(jnp.where(sel, cy1v, -1e30))
            x2i = jnp.max(jnp.where(sel, cx2v, -1e30))
            y2i = jnp.max(jnp.where(sel, cy2v, -1e30))
            ai = jnp.max(jnp.where(sel, areasv, -1e30))
            xx1 = jnp.maximum(x1i, cx1v)
            yy1 = jnp.maximum(y1i, cy1v)
            xx2 = jnp.minimum(x2i, cx2v)
            yy2 = jnp.minimum(y2i, cy2v)
            ww = jnp.maximum(xx2 - xx1 + 1.0, 0.0)
            hh = jnp.maximum(yy2 - yy1 + 1.0, 0.0)
            inter = ww * hh
            iou = inter / (ai + areasv - inter)
            sup = (iou > _NMS_THRESH) & (lane > i)
            keep[...] = jnp.where((ki > 0) & sup, 0, keep[...])
            return carry

        jax.lax.fori_loop(0, pc, nms_body, (cx1, cy1, cx2, cy2, areas))

    fv = jnp.where(
        lane < _PRE,
        jnp.where((bs[...] > _SCORE_THRESH) & (keep[...] > 0), bs[...], -1.0),
        -2.0)

    unsortedf = _num_unsorted_desc(fv)

    @pl.when(unsortedf == 0.0)
    def _():
        ffs[...] = fv
        ffl[...] = bi[...]

    @pl.when(unsortedf != 0.0)
    def _():
        fs2, _, (fl2,) = _bitonic_sort(fv, lane, [bi[...]])
        ffs[...] = fs2
        ffl[...] = fl2

    osc_ref[...] = ffs[0:1, 0:128]
    lab = (ffl[...] % _NFG) + 1
    olb_ref[...] = lab[0:1, 0:128]

    cp2 = pltpu.make_async_copy(ffl.at[0:1, :], ffl_sm, sem)
    cp2.start()
    cp2.wait()

    def issue_out(j, _):
        flat = ffl_sm[0, j]
        n = flat // _NFG
        cc = flat % _NFG
        base = pl.multiple_of(128 * ((4 * cc) // 128), 128)
        pltpu.make_async_copy(
            boxes_ref.at[pl.ds(n, 1), pl.ds(base, 128)],
            obox8.at[pl.ds(j, 1), :], sem).start()
        return 0

    jax.lax.fori_loop(0, _DET, issue_out, 0)

    def drain_out(j, _):
        pltpu.make_async_copy(
            boxes_ref.at[pl.ds(0, 1), pl.ds(0, 128)],
            obox8.at[pl.ds(0, 1), :], sem).wait()
        return 0

    jax.lax.fori_loop(0, _DET, drain_out, 0)

    qof = ((4 * (ffl[...] % _NFG)) % 128).astype(jnp.float32)
    qofT = jnp.transpose(qof[0:1, 0:128])
    colo = jax.lax.broadcasted_iota(
        jnp.int32, (128, 128), 1).astype(jnp.float32)
    vals = [jnp.sum(jnp.where(colo == qofT + p, obox8[...], 0.0),
                    axis=1, keepdims=True) for p in range(4)]
    obox_ref[...] = jnp.concatenate(vals, axis=1)


@jax.jit
def kernel(x, proposals, W1, b1, W2, b2, Wc, bc, Wb, bb):
    xp = jnp.pad(x, ((0, _NPAD - _N), (0, 0))).astype(jnp.bfloat16)
    pp = jnp.pad(proposals, ((0, _NPAD - _N), (0, 0)))
    w1b = W1.astype(jnp.bfloat16)
    w2b = W2.astype(jnp.bfloat16)
    wcp = jnp.pad(Wc, ((0, 0), (0, 128 - _NCLS))).astype(jnp.bfloat16)
    bcp = jnp.pad(bc, ((0, 128 - _NCLS),)).reshape(1, 128)
    wbp = jnp.pad(Wb.reshape(1024, _NCLS, 4).transpose(0, 2, 1),
                  ((0, 0), (0, 0), (0, 128 - _NCLS))
                  ).reshape(1024, 512).astype(jnp.bfloat16)
    bbp = jnp.pad(bb.reshape(_NCLS, 4).T,
                  ((0, 0), (0, 128 - _NCLS))).reshape(1, 512)
    src = jax.lax.broadcasted_iota(jnp.int32, (320, 384), 0)
    dst = jax.lax.broadcasted_iota(jnp.int32, (320, 384), 1)
    perm = ((dst < 320) & ((dst % 4) * _NFG + dst // 4 == src)
            ).astype(jnp.float32)

    scores, boxes = pl.pallas_call(
        _head_body,
        grid=(_NTILES,),
        in_specs=[
            pl.BlockSpec((_TN, 256), lambda i: (i, 0)),
            pl.BlockSpec((_TN, 4), lambda i: (i, 0)),
            pl.BlockSpec((256, 1024), lambda i: (0, 0)),
            pl.BlockSpec((1, 1024), lambda i: (0, 0)),
            pl.BlockSpec((1024, 1024), lambda i: (0, 0)),
            pl.BlockSpec((1, 1024), lambda i: (0, 0)),
            pl.BlockSpec((1024, 128), lambda i: (0, 0)),
            pl.BlockSpec((1, 128), lambda i: (0, 0)),
            pl.BlockSpec((1024, 512), lambda i: (0, 0)),
            pl.BlockSpec((1, 512), lambda i: (0, 0)),
            pl.BlockSpec((320, 384), lambda i: (0, 0)),
        ],
        out_specs=[
            pl.BlockSpec((_TN, _NFG), lambda i: (i, 0)),
            pl.BlockSpec((_TN, 384), lambda i: (i, 0)),
        ],
        out_shape=[
            jax.ShapeDtypeStruct((_NPAD, _NFG), jnp.float32),
            jax.ShapeDtypeStruct((_NPAD, 384), jnp.float32),
        ],
        compiler_params=pltpu.CompilerParams(
            dimension_semantics=("arbitrary",)),
        interpret=_INTERPRET,
    )(xp, pp, w1b, b1.reshape(1, 1024), w2b, b2.reshape(1, 1024),
      wcp, bcp, wbp, bbp, perm)

    s2 = scores.reshape(_NCHUNK, _CH)

    obox, osc, olb = pl.pallas_call(
        _post_body,
        in_specs=[
            pl.BlockSpec(memory_space=pltpu.VMEM),
            pl.BlockSpec(memory_space=pltpu.HBM),
        ],
        out_specs=[
            pl.BlockSpec(memory_space=pltpu.VMEM),
            pl.BlockSpec(memory_space=pltpu.VMEM),
            pl.BlockSpec(memory_space=pltpu.VMEM),
        ],
        out_shape=[
            jax.ShapeDtypeStruct((128, 4), jnp.float32),
            jax.ShapeDtypeStruct((1, 128), jnp.float32),
            jax.ShapeDtypeStruct((1, 128), jnp.int32),
        ],
        scratch_shapes=[
            pltpu.VMEM((_R, _CH), jnp.float32),
            pltpu.VMEM((_R, _CH), jnp.int32),
            pltpu.VMEM((_R, _CH), jnp.int32),
            pltpu.VMEM((_CH, 128), jnp.float32),
            pltpu.VMEM((128, 128), jnp.float32),
            pltpu.VMEM((_R, _CH), jnp.float32),
            pltpu.VMEM((_R, _CH), jnp.int32),
            pltpu.SMEM((1, _CH), jnp.int32),
            pltpu.SMEM((1, _CH), jnp.int32),
            pltpu.SemaphoreType.DMA,
        ],
        interpret=_INTERPRET,
    )(s2, boxes)

    det_boxes = obox[:_DET, :]
    det_scores = osc[0, :_DET]
    det_labels = olb[0, :_DET]
    return det_boxes, det_scores, det_labels


# speedup vs baseline: 39.7496x; 1.0968x over previous
import math

import jax
import jax.numpy as jnp
from jax.experimental import pallas as pl
from jax.experimental.pallas import tpu as pltpu

_SCORE_THRESH = 0.05
_NMS_THRESH = 0.5
_DET = 100
_NCLS = 81
_NFG = 80
_PRE = 2000
_IMG_H, _IMG_W = 800.0, 1333.0
_CLIP = math.log(1000.0 / 16.0)

_N = 20000
_NPAD = 20480
_TN = 512
_NTILES = _NPAD // _TN
_CH = 2048
_NCHUNK = (_NPAD * _NFG) // _CH
_R = 8

_INTERPRET = False
_MMDT = jnp.bfloat16


def _rep(x):
    return jnp.broadcast_to(x, (_R,) + x.shape[1:])


def _cmp_stage(lane, k, j, s, i, payloads, descending):
    pplus = (lane & j) == 0
    def part(a):
        return jnp.where(pplus, jnp.roll(a, -j, axis=1), jnp.roll(a, j, axis=1))
    ps, pi = part(s), part(i)
    better = (s > ps) | ((s == ps) & (i < pi))
    if not descending:
        better = ~better
    dirblock = (lane & k) == 0
    keep_self = dirblock ^ pplus ^ better
    s2 = jnp.where(keep_self, s, ps)
    i2 = jnp.where(keep_self, i, pi)
    pay2 = [jnp.where(keep_self, p, part(p)) for p in payloads]
    return s2, i2, pay2


def _bitonic_sort(s, i, payloads, descending=True):
    n = s.shape[1]
    lane = jax.lax.broadcasted_iota(jnp.int32, (_R, n), 1)
    k = 2
    while k <= n:
        j = k // 2
        while j >= 1:
            s, i, payloads = _cmp_stage(lane, k, j, s, i, payloads, descending)
            j //= 2
        k *= 2
    return s, i, payloads


def _bitonic_merge_desc(s, i, payloads):
    n = s.shape[1]
    lane = jax.lax.broadcasted_iota(jnp.int32, (_R, n), 1)
    j = n // 2
    while j >= 1:
        s, i, payloads = _cmp_stage(lane, n, j, s, i, payloads, True)
        j //= 2
    return s, i, payloads


def _num_unsorted_desc(s):
    return jnp.sum(jnp.where(s[:, :-1] < s[:, 1:], 1.0, 0.0))


def _head_body(x_ref, p_ref, w1_ref, b1_ref, w2_ref, b2_ref, wc_ref, bc_ref,
               wb_ref, bb_ref, s_ref, bx_ref):
    f1 = jnp.maximum(
        jnp.dot(x_ref[...], w1_ref[...], preferred_element_type=jnp.float32)
        + b1_ref[...], 0.0)
    f1b = f1.astype(_MMDT)
    f2 = jnp.maximum(
        jnp.dot(f1b, w2_ref[...], preferred_element_type=jnp.float32)
        + b2_ref[...], 0.0)
    f2b = f2.astype(_MMDT)
    logits = (jnp.dot(f2b, wc_ref[...], preferred_element_type=jnp.float32)
              + bc_ref[...])
    col = jax.lax.broadcasted_iota(jnp.int32, (_TN, 128), 1)
    logits = jnp.where(col < _NCLS, logits, -1e30)
    m = jnp.max(logits, axis=-1, keepdims=True)
    e = jnp.exp(logits - m)
    s_sum = jnp.sum(e, axis=-1, keepdims=True)
    posfg = ((e > (0.999 * _SCORE_THRESH) * s_sum)
             & (col >= 1) & (col < _NCLS))
    npos = jnp.sum(jnp.where(posfg, 1.0, 0.0))

    breg = (jnp.dot(f2b, wb_ref[...], preferred_element_type=jnp.float32)
            + bb_ref[...])
    dx = breg[:, 1:81] / 10.0
    dy = breg[:, 129:209] / 10.0
    dw = jnp.minimum(breg[:, 257:337] / 5.0, _CLIP)
    dh = jnp.minimum(breg[:, 385:465] / 5.0, _CLIP)

    px1 = p_ref[:, 0:1]
    py1 = p_ref[:, 1:2]
    px2 = p_ref[:, 2:3]
    py2 = p_ref[:, 3:4]
    w = px2 - px1 + 1.0
    h = py2 - py1 + 1.0
    cx = px1 + 0.5 * w
    cy = py1 + 0.5 * h
    pcx = dx * w + cx
    pcy = dy * h + cy
    pw = jnp.exp(dw) * w
    ph = jnp.exp(dh) * h
    bx1 = jnp.clip(pcx - 0.5 * pw, 0.0, _IMG_W - 1.0)
    by1 = jnp.clip(pcy - 0.5 * ph, 0.0, _IMG_H - 1.0)
    bx2 = jnp.clip(pcx + 0.5 * pw - 1.0, 0.0, _IMG_W - 1.0)
    by2 = jnp.clip(pcy + 0.5 * ph - 1.0, 0.0, _IMG_H - 1.0)

    row = (jax.lax.broadcasted_iota(jnp.int32, (_TN, _NFG), 0)
           + pl.program_id(0) * _TN)
    base = jnp.where(row < _N, -1.0, -2.0)
    s_ref[...] = base

    @pl.when(npos > 0.0)
    def _():
        sm = e / s_sum
        sfg = sm[:, 1:_NCLS]
        s_ref[...] = jnp.where((sfg > _SCORE_THRESH) & (row < _N), sfg, base)

    bx_ref[:, 0:80] = bx1
    bx_ref[:, 128:208] = by1
    bx_ref[:, 256:336] = bx2
    bx_ref[:, 384:464] = by2


def _post_body(s_ref, boxes_ref, obox_ref, osc_ref, olb_ref,
               bs, bi, keep, rows4, obox8, blk8, ffs, ffl, ffl_sm, bi_sm,
               sem):
    lane = jax.lax.broadcasted_iota(jnp.int32, (_R, _CH), 1)
    c0 = _rep(s_ref[0:1, :])

    unsorted0 = _num_unsorted_desc(c0)

    @pl.when(unsorted0 == 0.0)
    def _():
        bs[...] = c0
        bi[...] = lane

    @pl.when(unsorted0 != 0.0)
    def _():
        ss, ii, _ = _bitonic_sort(c0, lane, [])
        bs[...] = ss
        bi[...] = ii

    bmin0 = bs[0, _CH - 1]
    full_max = jnp.max(s_ref[...])

    @pl.when(full_max > bmin0)
    def _():
        def merge_body(c, _):
            ch = _rep(s_ref[pl.ds(c, 1), :])
            m = jnp.max(ch)

            @pl.when(m > bmin0)
            def _():
                ci = lane + c * _CH
                sa, ia, _ = _bitonic_sort(ch, ci, [], descending=False)
                cs = jnp.concatenate([bs[...], sa], axis=1)
                cidx = jnp.concatenate([bi[...], ia], axis=1)
                ms, mi, _ = _bitonic_merge_desc(cs, cidx, [])
                bs[...] = ms[:, :_CH]
                bi[...] = mi[:, :_CH]
            return 0

        jax.lax.fori_loop(1, _NCHUNK, merge_body, 0)

    pc_f = jnp.sum(jnp.where(
        (lane < _PRE) & (bs[...] > _SCORE_THRESH), 1.0, 0.0)) / _R
    pc = pc_f.astype(jnp.int32)

    keep[...] = jnp.ones((_R, _CH), jnp.int32)

    @pl.when(pc > 0)
    def _():
        rows4[...] = jnp.zeros((_CH, 512), jnp.float32)
        cp = pltpu.make_async_copy(bi.at[0:1, :], bi_sm, sem)
        cp.start()
        cp.wait()

        def issue(j, _):
            flat = bi_sm[0, j]
            n = flat // _NFG
            pltpu.make_async_copy(
                boxes_ref.at[pl.ds(n, 1), :],
                rows4.at[pl.ds(j, 1), :], sem).start()
            return 0

        jax.lax.fori_loop(0, pc, issue, 0)

        def drain(j, _):
            pltpu.make_async_copy(
                boxes_ref.at[pl.ds(0, 1), :],
                rows4.at[pl.ds(0, 1), :], sem).wait()
            return 0

        jax.lax.fori_loop(0, pc, drain, 0)

        qo = (bi[...] % _NFG).astype(jnp.float32)
        qoT = jnp.transpose(qo[0:1, :])
        colw = jax.lax.broadcasted_iota(
            jnp.int32, (_CH, 128), 1).astype(jnp.float32)
        selm = colw == qoT

        def coord(p):
            v = jnp.sum(
                jnp.where(selm, rows4[:, 128 * p:128 * (p + 1)], 0.0),
                axis=1, keepdims=True)
            return _rep(jnp.transpose(v))

        off = ((bi[...] % _NFG) + 1).astype(jnp.float32) * (_IMG_W + _IMG_H)
        cx1 = coord(0) + off
        cy1 = coord(1) + off
        cx2 = coord(2) + off
        cy2 = coord(3) + off
        areas = (cx2 - cx1 + 1.0) * (cy2 - cy1 + 1.0)

        def nms_body(i, carry):
            cx1v, cy1v, cx2v, cy2v, areasv = carry
            sel = (lane == i)
            ki = jnp.max(jnp.where(sel, keep[...], 0))
            x1i = jnp.max(jnp.where(sel, cx1v, -1e30))
            y1i = jnp.max(jnp.where(sel, cy1v, -1e30))
            x2i = jnp.max(jnp.where(sel, cx2v, -1e30))
            y2i = jnp.max(jnp.where(sel, cy2v, -1e30))
            ai = jnp.max(jnp.where(sel, areasv, -1e30))
            xx1 = jnp.maximum(x1i, cx1v)
            yy1 = jnp.maximum(y1i, cy1v)
            xx2 = jnp.minimum(x2i, cx2v)
            yy2 = jnp.minimum(y2i, cy2v)
            ww = jnp.maximum(xx2 - xx1 + 1.0, 0.0)
            hh = jnp.maximum(yy2 - yy1 + 1.0, 0.0)
            inter = ww * hh
            iou = inter / (ai + areasv - inter)
            sup = (iou > _NMS_THRESH) & (lane > i)
            keep[...] = jnp.where((ki > 0) & sup, 0, keep[...])
            return carry

        jax.lax.fori_loop(0, pc, nms_body, (cx1, cy1, cx2, cy2, areas))

    fv = jnp.where(
        lane < _PRE,
        jnp.where((bs[...] > _SCORE_THRESH) & (keep[...] > 0), bs[...], -1.0),
        -2.0)

    unsortedf = _num_unsorted_desc(fv)

    @pl.when(unsortedf == 0.0)
    def _():
        ffs[...] = fv
        ffl[...] = bi[...]

    @pl.when(unsortedf != 0.0)
    def _():
        fs2, _, (fl2,) = _bitonic_sort(fv, lane, [bi[...]])
        ffs[...] = fs2
        ffl[...] = fl2

    osc_ref[...] = ffs[0:1, 0:128]
    lab = (ffl[...] % _NFG) + 1
    olb_ref[...] = lab[0:1, 0:128]

    lm = lane < _DET
    nsel = ffl[...] // _NFG
    nminn = jnp.min(jnp.where(lm, nsel, jnp.int32(1 << 30)))
    nmaxn = jnp.max(jnp.where(lm, nsel, -1))
    nbase = pl.multiple_of(
        jnp.minimum((nminn // 8) * 8, _NPAD - 16), 8)

    @pl.when(nmaxn - nbase < 16)
    def _():
        cpb = pltpu.make_async_copy(
            boxes_ref.at[pl.ds(nbase, 16), :], blk8, sem)
        cpb.start()
        cpb.wait()
        rT = jnp.transpose((nsel[0:1, 0:128] - nbase).astype(jnp.float32))
        rio = jax.lax.broadcasted_iota(
            jnp.int32, (128, 16), 1).astype(jnp.float32)
        rowsel = (rio == rT).astype(jnp.float32)
        obox8[...] = jnp.dot(rowsel, blk8[...],
                             preferred_element_type=jnp.float32)

    @pl.when(nmaxn - nbase >= 16)
    def _():
        cp2 = pltpu.make_async_copy(ffl.at[0:1, :], ffl_sm, sem)
        cp2.start()
        cp2.wait()

        def issue_out(j, _):
            flat = ffl_sm[0, j]
            n = flat // _NFG
            pltpu.make_async_copy(
                boxes_ref.at[pl.ds(n, 1), :],
                obox8.at[pl.ds(j, 1), :], sem).start()
            return 0

        jax.lax.fori_loop(0, _DET, issue_out, 0)

        def drain_out(j, _):
            pltpu.make_async_copy(
                boxes_ref.at[pl.ds(0, 1), :],
                obox8.at[pl.ds(0, 1), :], sem).wait()
            return 0

        jax.lax.fori_loop(0, _DET, drain_out, 0)

    qofT = jnp.transpose((ffl[0:1, 0:128] % _NFG).astype(jnp.float32))
    colo = jax.lax.broadcasted_iota(
        jnp.int32, (128, 128), 1).astype(jnp.float32)
    colsel = colo == qofT
    vals = [jnp.sum(jnp.where(colsel, obox8[:, 128 * p:128 * (p + 1)], 0.0),
                    axis=1, keepdims=True) for p in range(4)]
    obox_ref[...] = jnp.concatenate(vals, axis=1)


@jax.jit
def kernel(x, proposals, W1, b1, W2, b2, Wc, bc, Wb, bb):
    xp = jnp.pad(x, ((0, _NPAD - _N), (0, 0))).astype(_MMDT)
    pp = jnp.pad(proposals, ((0, _NPAD - _N), (0, 0)))
    w1b = W1.astype(_MMDT)
    w2b = W2.astype(_MMDT)
    wcp = jnp.pad(Wc, ((0, 0), (0, 128 - _NCLS))).astype(_MMDT)
    bcp = jnp.pad(bc, ((0, 128 - _NCLS),)).reshape(1, 128)
    wbp = jnp.pad(Wb.reshape(1024, _NCLS, 4).transpose(0, 2, 1),
                  ((0, 0), (0, 0), (0, 128 - _NCLS))
                  ).reshape(1024, 512).astype(_MMDT)
    bbp = jnp.pad(bb.reshape(_NCLS, 4).T,
                  ((0, 0), (0, 128 - _NCLS))).reshape(1, 512)

    scores, boxes = pl.pallas_call(
        _head_body,
        grid=(_NTILES,),
        in_specs=[
            pl.BlockSpec((_TN, 256), lambda i: (i, 0)),
            pl.BlockSpec((_TN, 4), lambda i: (i, 0)),
            pl.BlockSpec((256, 1024), lambda i: (0, 0)),
            pl.BlockSpec((1, 1024), lambda i: (0, 0)),
            pl.BlockSpec((1024, 1024), lambda i: (0, 0)),
            pl.BlockSpec((1, 1024), lambda i: (0, 0)),
            pl.BlockSpec((1024, 128), lambda i: (0, 0)),
            pl.BlockSpec((1, 128), lambda i: (0, 0)),
            pl.BlockSpec((1024, 512), lambda i: (0, 0)),
            pl.BlockSpec((1, 512), lambda i: (0, 0)),
        ],
        out_specs=[
            pl.BlockSpec((_TN, _NFG), lambda i: (i, 0)),
            pl.BlockSpec((_TN, 512), lambda i: (i, 0)),
        ],
        out_shape=[
            jax.ShapeDtypeStruct((_NPAD, _NFG), jnp.float32),
            jax.ShapeDtypeStruct((_NPAD, 512), jnp.float32),
        ],
        compiler_params=pltpu.CompilerParams(
            dimension_semantics=("arbitrary",)),
        interpret=_INTERPRET,
    )(xp, pp, w1b, b1.reshape(1, 1024), w2b, b2.reshape(1, 1024),
      wcp, bcp, wbp, bbp)

    s2 = scores.reshape(_NCHUNK, _CH)

    obox, osc, olb = pl.pallas_call(
        _post_body,
        in_specs=[
            pl.BlockSpec(memory_space=pltpu.VMEM),
            pl.BlockSpec(memory_space=pltpu.HBM),
        ],
        out_specs=[
            pl.BlockSpec(memory_space=pltpu.VMEM),
            pl.BlockSpec(memory_space=pltpu.VMEM),
            pl.BlockSpec(memory_space=pltpu.VMEM),
        ],
        out_shape=[
            jax.ShapeDtypeStruct((128, 4), jnp.float32),
            jax.ShapeDtypeStruct((1, 128), jnp.float32),
            jax.ShapeDtypeStruct((1, 128), jnp.int32),
        ],
        scratch_shapes=[
            pltpu.VMEM((_R, _CH), jnp.float32),
            pltpu.VMEM((_R, _CH), jnp.int32),
            pltpu.VMEM((_R, _CH), jnp.int32),
            pltpu.VMEM((_CH, 512), jnp.float32),
            pltpu.VMEM((128, 512), jnp.float32),
            pltpu.VMEM((16, 512), jnp.float32),
            pltpu.VMEM((_R, _CH), jnp.float32),
            pltpu.VMEM((_R, _CH), jnp.int32),
            pltpu.SMEM((1, _CH), jnp.int32),
            pltpu.SMEM((1, _CH), jnp.int32),
            pltpu.SemaphoreType.DMA,
        ],
        interpret=_INTERPRET,
    )(s2, boxes)

    det_boxes = obox[:_DET, :]
    det_scores = osc[0, :_DET]
    det_labels = olb[0, :_DET]
    return det_boxes, det_scores, det_labels


# speedup vs baseline: 44.3721x; 1.1163x over previous
import math

import jax
import jax.numpy as jnp
from jax.experimental import pallas as pl
from jax.experimental.pallas import tpu as pltpu

_SCORE_THRESH = 0.05
_NMS_THRESH = 0.5
_DET = 100
_NCLS = 81
_NFG = 80
_PRE = 2000
_IMG_H, _IMG_W = 800.0, 1333.0
_CLIP = math.log(1000.0 / 16.0)

_N = 20000
_NPAD = 20480
_TN = 512
_NTILES = _NPAD // _TN
_CH = 2048
_NCHUNK = (_NPAD * _NFG) // _CH
_R = 8

_INTERPRET = False
_MMDT = jnp.bfloat16


def _rep(x):
    return jnp.broadcast_to(x, (_R,) + x.shape[1:])


def _cmp_stage(lane, k, j, s, i, payloads, descending):
    pplus = (lane & j) == 0
    def part(a):
        return jnp.where(pplus, jnp.roll(a, -j, axis=1), jnp.roll(a, j, axis=1))
    ps, pi = part(s), part(i)
    better = (s > ps) | ((s == ps) & (i < pi))
    if not descending:
        better = ~better
    dirblock = (lane & k) == 0
    keep_self = dirblock ^ pplus ^ better
    s2 = jnp.where(keep_self, s, ps)
    i2 = jnp.where(keep_self, i, pi)
    pay2 = [jnp.where(keep_self, p, part(p)) for p in payloads]
    return s2, i2, pay2


def _bitonic_sort(s, i, payloads, descending=True):
    n = s.shape[1]
    lane = jax.lax.broadcasted_iota(jnp.int32, (_R, n), 1)
    k = 2
    while k <= n:
        j = k // 2
        while j >= 1:
            s, i, payloads = _cmp_stage(lane, k, j, s, i, payloads, descending)
            j //= 2
        k *= 2
    return s, i, payloads


def _bitonic_merge_desc(s, i, payloads):
    n = s.shape[1]
    lane = jax.lax.broadcasted_iota(jnp.int32, (_R, n), 1)
    j = n // 2
    while j >= 1:
        s, i, payloads = _cmp_stage(lane, n, j, s, i, payloads, True)
        j //= 2
    return s, i, payloads


def _num_unsorted_desc(s):
    return jnp.sum(jnp.where(s[:, :-1] < s[:, 1:], 1.0, 0.0))


def _head_body(x_ref, p_ref, w1_ref, b1_ref, w2_ref, b2_ref, wc_ref, bc_ref,
               wb_ref, bb_ref, s_ref, bx_ref):
    f1 = jnp.maximum(
        jnp.dot(x_ref[...].astype(_MMDT), w1_ref[...],
                preferred_element_type=jnp.float32)
        + b1_ref[...], 0.0)
    f1b = f1.astype(_MMDT)
    f2 = jnp.maximum(
        jnp.dot(f1b, w2_ref[...], preferred_element_type=jnp.float32)
        + b2_ref[...], 0.0)
    f2b = f2.astype(_MMDT)
    logits = (jnp.dot(f2b, wc_ref[...], preferred_element_type=jnp.float32)
              + bc_ref[...])
    col = jax.lax.broadcasted_iota(jnp.int32, (_TN, 128), 1)
    logits = jnp.where(col < _NCLS, logits, -1e30)
    m = jnp.max(logits, axis=-1, keepdims=True)
    e = jnp.exp(logits - m)
    s_sum = jnp.sum(e, axis=-1, keepdims=True)
    posfg = ((e > (0.999 * _SCORE_THRESH) * s_sum)
             & (col >= 1) & (col < _NCLS))
    npos = jnp.sum(jnp.where(posfg, 1.0, 0.0))

    breg = (jnp.dot(f2b, wb_ref[...], preferred_element_type=jnp.float32)
            + bb_ref[...])
    dx = breg[:, 1:81] / 10.0
    dy = breg[:, 129:209] / 10.0
    dw = jnp.minimum(breg[:, 257:337] / 5.0, _CLIP)
    dh = jnp.minimum(breg[:, 385:465] / 5.0, _CLIP)

    px1 = p_ref[:, 0:1]
    py1 = p_ref[:, 1:2]
    px2 = p_ref[:, 2:3]
    py2 = p_ref[:, 3:4]
    w = px2 - px1 + 1.0
    h = py2 - py1 + 1.0
    cx = px1 + 0.5 * w
    cy = py1 + 0.5 * h
    pcx = dx * w + cx
    pcy = dy * h + cy
    pw = jnp.exp(dw) * w
    ph = jnp.exp(dh) * h
    bx1 = jnp.clip(pcx - 0.5 * pw, 0.0, _IMG_W - 1.0)
    by1 = jnp.clip(pcy - 0.5 * ph, 0.0, _IMG_H - 1.0)
    bx2 = jnp.clip(pcx + 0.5 * pw - 1.0, 0.0, _IMG_W - 1.0)
    by2 = jnp.clip(pcy + 0.5 * ph - 1.0, 0.0, _IMG_H - 1.0)

    row = (jax.lax.broadcasted_iota(jnp.int32, (_TN, _NFG), 0)
           + pl.program_id(0) * _TN)
    base = jnp.where(row < _N, -1.0, -2.0)
    s_ref[...] = base

    @pl.when(npos > 0.0)
    def _():
        sm = e / s_sum
        sfg = sm[:, 1:_NCLS]
        s_ref[...] = jnp.where((sfg > _SCORE_THRESH) & (row < _N), sfg, base)

    bx_ref[:, 0:80] = bx1
    bx_ref[:, 128:208] = by1
    bx_ref[:, 256:336] = bx2
    bx_ref[:, 384:464] = by2


def _post_body(s_ref, boxes_ref, obox_ref, osc_ref, olb_ref,
               bs, bi, keep, rows4, obox8, blk8, ffs, ffl, ffl_sm, bi_sm,
               sem):
    lane = jax.lax.broadcasted_iota(jnp.int32, (_R, _CH), 1)
    c0 = _rep(s_ref[0:1, :])

    unsorted0 = _num_unsorted_desc(c0)

    @pl.when(unsorted0 == 0.0)
    def _():
        bs[...] = c0
        bi[...] = lane

    @pl.when(unsorted0 != 0.0)
    def _():
        ss, ii, _ = _bitonic_sort(c0, lane, [])
        bs[...] = ss
        bi[...] = ii

    bmin0 = bs[0, _CH - 1]
    full_max = jnp.max(s_ref[...])

    @pl.when(full_max > bmin0)
    def _():
        def merge_body(c, _):
            ch = _rep(s_ref[pl.ds(c, 1), :])
            m = jnp.max(ch)

            @pl.when(m > bmin0)
            def _():
                ci = lane + c * _CH
                sa, ia, _ = _bitonic_sort(ch, ci, [], descending=False)
                cs = jnp.concatenate([bs[...], sa], axis=1)
                cidx = jnp.concatenate([bi[...], ia], axis=1)
                ms, mi, _ = _bitonic_merge_desc(cs, cidx, [])
                bs[...] = ms[:, :_CH]
                bi[...] = mi[:, :_CH]
            return 0

        jax.lax.fori_loop(1, _NCHUNK, merge_body, 0)

    pc_f = jnp.sum(jnp.where(
        (lane < _PRE) & (bs[...] > _SCORE_THRESH), 1.0, 0.0)) / _R
    pc = pc_f.astype(jnp.int32)

    keep[...] = jnp.ones((_R, _CH), jnp.int32)

    @pl.when(pc > 0)
    def _():
        rows4[...] = jnp.zeros((_CH, 512), jnp.float32)
        cp = pltpu.make_async_copy(bi.at[0:1, :], bi_sm, sem)
        cp.start()
        cp.wait()

        def issue(j, _):
            flat = bi_sm[0, j]
            n = flat // _NFG
            pltpu.make_async_copy(
                boxes_ref.at[pl.ds(n, 1), :],
                rows4.at[pl.ds(j, 1), :], sem).start()
            return 0

        jax.lax.fori_loop(0, pc, issue, 0)

        def drain(j, _):
            pltpu.make_async_copy(
                boxes_ref.at[pl.ds(0, 1), :],
                rows4.at[pl.ds(0, 1), :], sem).wait()
            return 0

        jax.lax.fori_loop(0, pc, drain, 0)

        qo = (bi[...] % _NFG).astype(jnp.float32)
        qoT = jnp.transpose(qo[0:1, :])
        colw = jax.lax.broadcasted_iota(
            jnp.int32, (_CH, 128), 1).astype(jnp.float32)
        selm = colw == qoT

        def coord(p):
            v = jnp.sum(
                jnp.where(selm, rows4[:, 128 * p:128 * (p + 1)], 0.0),
                axis=1, keepdims=True)
            return _rep(jnp.transpose(v))

        off = ((bi[...] % _NFG) + 1).astype(jnp.float32) * (_IMG_W + _IMG_H)
        cx1 = coord(0) + off
        cy1 = coord(1) + off
        cx2 = coord(2) + off
        cy2 = coord(3) + off
        areas = (cx2 - cx1 + 1.0) * (cy2 - cy1 + 1.0)

        def nms_body(i, carry):
            cx1v, cy1v, cx2v, cy2v, areasv = carry
            sel = (lane == i)
            ki = jnp.max(jnp.where(sel, keep[...], 0))
            x1i = jnp.max(jnp.where(sel, cx1v, -1e30))
            y1i = jnp.max(jnp.where(sel, cy1v, -1e30))
            x2i = jnp.max(jnp.where(sel, cx2v, -1e30))
            y2i = jnp.max(jnp.where(sel, cy2v, -1e30))
            ai = jnp.max(jnp.where(sel, areasv, -1e30))
            xx1 = jnp.maximum(x1i, cx1v)
            yy1 = jnp.maximum(y1i, cy1v)
            xx2 = jnp.minimum(x2i, cx2v)
            yy2 = jnp.minimum(y2i, cy2v)
            ww = jnp.maximum(xx2 - xx1 + 1.0, 0.0)
            hh = jnp.maximum(yy2 - yy1 + 1.0, 0.0)
            inter = ww * hh
            iou = inter / (ai + areasv - inter)
            sup = (iou > _NMS_THRESH) & (lane > i)
            keep[...] = jnp.where((ki > 0) & sup, 0, keep[...])
            return carry

        jax.lax.fori_loop(0, pc, nms_body, (cx1, cy1, cx2, cy2, areas))

    fv = jnp.where(
        lane < _PRE,
        jnp.where((bs[...] > _SCORE_THRESH) & (keep[...] > 0), bs[...], -1.0),
        -2.0)

    unsortedf = _num_unsorted_desc(fv)

    @pl.when(unsortedf == 0.0)
    def _():
        ffs[...] = fv
        ffl[...] = bi[...]

    @pl.when(unsortedf != 0.0)
    def _():
        fs2, _, (fl2,) = _bitonic_sort(fv, lane, [bi[...]])
        ffs[...] = fs2
        ffl[...] = fl2

    osc_ref[...] = ffs[0:1, 0:128]
    lab = (ffl[...] % _NFG) + 1
    olb_ref[...] = lab[0:1, 0:128]

    lm = lane < _DET
    nsel = ffl[...] // _NFG
    nminn = jnp.min(jnp.where(lm, nsel, jnp.int32(1 << 30)))
    nmaxn = jnp.max(jnp.where(lm, nsel, -1))
    nbase = pl.multiple_of(
        jnp.minimum((nminn // 8) * 8, _NPAD - 16), 8)

    @pl.when(nmaxn - nbase < 16)
    def _():
        cpb = pltpu.make_async_copy(
            boxes_ref.at[pl.ds(nbase, 16), :], blk8, sem)
        cpb.start()
        cpb.wait()
        rT = jnp.transpose((nsel[0:1, 0:128] - nbase).astype(jnp.float32))
        rio = jax.lax.broadcasted_iota(
            jnp.int32, (128, 16), 1).astype(jnp.float32)
        rowsel = (rio == rT).astype(jnp.float32)
        obox8[...] = jnp.dot(rowsel, blk8[...],
                             preferred_element_type=jnp.float32)

    @pl.when(nmaxn - nbase >= 16)
    def _():
        cp2 = pltpu.make_async_copy(ffl.at[0:1, :], ffl_sm, sem)
        cp2.start()
        cp2.wait()

        def issue_out(j, _):
            flat = ffl_sm[0, j]
            n = flat // _NFG
            pltpu.make_async_copy(
                boxes_ref.at[pl.ds(n, 1), :],
                obox8.at[pl.ds(j, 1), :], sem).start()
            return 0

        jax.lax.fori_loop(0, _DET, issue_out, 0)

        def drain_out(j, _):
            pltpu.make_async_copy(
                boxes_ref.at[pl.ds(0, 1), :],
                obox8.at[pl.ds(0, 1), :], sem).wait()
            return 0

        jax.lax.fori_loop(0, _DET, drain_out, 0)

    qofT = jnp.transpose((ffl[0:1, 0:128] % _NFG).astype(jnp.float32))
    colo = jax.lax.broadcasted_iota(
        jnp.int32, (128, 128), 1).astype(jnp.float32)
    colsel = colo == qofT
    vals = [jnp.sum(jnp.where(colsel, obox8[:, 128 * p:128 * (p + 1)], 0.0),
                    axis=1, keepdims=True) for p in range(4)]
    obox_ref[...] = jnp.concatenate(vals, axis=1)


@jax.jit
def kernel(x, proposals, W1, b1, W2, b2, Wc, bc, Wb, bb):
    w1b = W1.astype(_MMDT)
    w2b = W2.astype(_MMDT)
    wcp = jnp.pad(Wc, ((0, 0), (0, 128 - _NCLS))).astype(_MMDT)
    bcp = jnp.pad(bc, ((0, 128 - _NCLS),)).reshape(1, 128)
    wbp = jnp.pad(Wb.reshape(1024, _NCLS, 4).transpose(0, 2, 1),
                  ((0, 0), (0, 0), (0, 128 - _NCLS))
                  ).reshape(1024, 512).astype(_MMDT)
    bbp = jnp.pad(bb.reshape(_NCLS, 4).T,
                  ((0, 0), (0, 128 - _NCLS))).reshape(1, 512)

    scores, boxes = pl.pallas_call(
        _head_body,
        grid=(_NTILES,),
        in_specs=[
            pl.BlockSpec((_TN, 256), lambda i: (i, 0)),
            pl.BlockSpec((_TN, 4), lambda i: (i, 0)),
            pl.BlockSpec((256, 1024), lambda i: (0, 0)),
            pl.BlockSpec((1, 1024), lambda i: (0, 0)),
            pl.BlockSpec((1024, 1024), lambda i: (0, 0)),
            pl.BlockSpec((1, 1024), lambda i: (0, 0)),
            pl.BlockSpec((1024, 128), lambda i: (0, 0)),
            pl.BlockSpec((1, 128), lambda i: (0, 0)),
            pl.BlockSpec((1024, 512), lambda i: (0, 0)),
            pl.BlockSpec((1, 512), lambda i: (0, 0)),
        ],
        out_specs=[
            pl.BlockSpec((_TN, _NFG), lambda i: (i, 0)),
            pl.BlockSpec((_TN, 512), lambda i: (i, 0)),
        ],
        out_shape=[
            jax.ShapeDtypeStruct((_NPAD, _NFG), jnp.float32),
            jax.ShapeDtypeStruct((_NPAD, 512), jnp.float32),
        ],
        compiler_params=pltpu.CompilerParams(
            dimension_semantics=("arbitrary",)),
        interpret=_INTERPRET,
    )(x, proposals, w1b, b1.reshape(1, 1024), w2b, b2.reshape(1, 1024),
      wcp, bcp, wbp, bbp)

    s2 = scores.reshape(_NCHUNK, _CH)

    obox, osc, olb = pl.pallas_call(
        _post_body,
        in_specs=[
            pl.BlockSpec(memory_space=pltpu.VMEM),
            pl.BlockSpec(memory_space=pltpu.HBM),
        ],
        out_specs=[
            pl.BlockSpec(memory_space=pltpu.VMEM),
            pl.BlockSpec(memory_space=pltpu.VMEM),
            pl.BlockSpec(memory_space=pltpu.VMEM),
        ],
        out_shape=[
            jax.ShapeDtypeStruct((128, 4), jnp.float32),
            jax.ShapeDtypeStruct((1, 128), jnp.float32),
            jax.ShapeDtypeStruct((1, 128), jnp.int32),
        ],
        scratch_shapes=[
            pltpu.VMEM((_R, _CH), jnp.float32),
            pltpu.VMEM((_R, _CH), jnp.int32),
            pltpu.VMEM((_R, _CH), jnp.int32),
            pltpu.VMEM((_CH, 512), jnp.float32),
            pltpu.VMEM((128, 512), jnp.float32),
            pltpu.VMEM((16, 512), jnp.float32),
            pltpu.VMEM((_R, _CH), jnp.float32),
            pltpu.VMEM((_R, _CH), jnp.int32),
            pltpu.SMEM((1, _CH), jnp.int32),
            pltpu.SMEM((1, _CH), jnp.int32),
            pltpu.SemaphoreType.DMA,
        ],
        interpret=_INTERPRET,
    )(s2, boxes)

    det_boxes = obox[:_DET, :]
    det_scores = osc[0, :_DET]
    det_labels = olb[0, :_DET]
    return det_boxes, det_scores, det_labels
